# baseline (speedup 1.0000x reference)
import jax
import jax.numpy as jnp
from jax.experimental import pallas as pl
from jax.experimental.pallas import tpu as pltpu

_M = 8192
_N = 8192
_FX = 1000.0 / 640.0
_FY = 1000.0 / 480.0
_ZOFF = 2.5
_CH = 4096


def _split_hi_lo(v):
    hi = v.astype(jnp.bfloat16)
    lo = (v - hi.astype(jnp.float32)).astype(jnp.bfloat16)
    return hi, lo


def _chamfer_body(inpT_ref, yT_ref, out_ref, at_ref):
    yT = yT_ref[...]
    yx = yT[0:1, :]
    yy = yT[1:2, :]
    s = yx * yx + yy * yy
    bxh, bxl = _split_hi_lo(yx)
    byh, byl = _split_hi_lo(yy)
    sh, sl = _split_hi_lo(s)
    ones_n = jnp.ones((1, _N), dtype=jnp.bfloat16)
    B = jnp.concatenate(
        [bxh, bxh, bxl, byh, byh, byl, ones_n, ones_n, sh, sl], axis=0)

    z = inpT_ref[2:3, :] + _ZOFF
    px = inpT_ref[0:1, :] * _FX / z
    py = inpT_ref[1:2, :] * _FY / z
    r = px * px + py * py
    axh, axl = _split_hi_lo(px * -2.0)
    ayh, ayl = _split_hi_lo(py * -2.0)
    rh, rl = _split_hi_lo(r)
    ones_m = jnp.ones((1, _M), dtype=jnp.bfloat16)
    at_ref[...] = jnp.concatenate(
        [axh, axl, axh, ayh, ayl, ayh, rh, rl, ones_m, ones_m], axis=0)

    def body(i, carry):
        rowsum, colmin = carry
        At_chunk = at_ref[:, pl.ds(i * _CH, _CH)]
        d2 = jax.lax.dot_general(
            At_chunk, B, (((0,), (0,)), ((), ())),
            preferred_element_type=jnp.float32)
        rowsum = rowsum + jnp.sum(jnp.min(d2, axis=1, keepdims=True))
        colmin = jnp.minimum(colmin, jnp.min(d2, axis=0, keepdims=True))
        return rowsum, colmin

    carry = (jnp.float32(0.0), jnp.full((1, _N), jnp.inf, dtype=jnp.float32))
    for i in range(_M // _CH):
        carry = body(i, carry)
    rowsum, colmin = carry
    out_ref[...] = jnp.reshape(rowsum / _M + jnp.sum(colmin) / _N, (1, 1))


@jax.jit
def kernel(input, mask_samples):
    inpT = input.T
    yT = mask_samples[0].T
    out = pl.pallas_call(
        _chamfer_body,
        out_shape=jax.ShapeDtypeStruct((1, 1), jnp.float32),
        scratch_shapes=[pltpu.VMEM((10, _M), jnp.bfloat16)],
    )(inpT, yT)
    return out[0, 0]


# speedup vs baseline: 1.1143x; 1.1143x over previous
import jax
import jax.numpy as jnp
from jax.experimental import pallas as pl
from jax.experimental.pallas import tpu as pltpu

_M = 8192
_N = 8192
_FX = 1000.0 / 640.0
_FY = 1000.0 / 480.0
_ZOFF = 2.5
_CH = 4096


def _split_hi_lo(v):
    hi = v.astype(jnp.bfloat16)
    lo = (v - hi.astype(jnp.float32)).astype(jnp.bfloat16)
    return hi, lo


def _chamfer_body(inpT_ref, yT_ref, out_ref, at_ref):
    yT = yT_ref[...]
    yx = yT[0:1, :]
    yy = yT[1:2, :]
    s = yx * yx + yy * yy
    bxh, bxl = _split_hi_lo(yx)
    byh, byl = _split_hi_lo(yy)
    sh, sl = _split_hi_lo(s)
    ones_n = jnp.ones((1, _N), dtype=jnp.bfloat16)
    B = jnp.concatenate(
        [bxh, bxh, bxl, byh, byh, byl, ones_n, ones_n, sh, sl], axis=0)

    z = inpT_ref[2:3, :] + _ZOFF
    px = inpT_ref[0:1, :] * _FX / z
    py = inpT_ref[1:2, :] * _FY / z
    r = px * px + py * py
    axh, axl = _split_hi_lo(px * -2.0)
    ayh, ayl = _split_hi_lo(py * -2.0)
    rh, rl = _split_hi_lo(r)
    ones_m = jnp.ones((1, _M), dtype=jnp.bfloat16)
    at_ref[...] = jnp.concatenate(
        [axh, axl, axh, ayh, ayl, ayh, rh, rl, ones_m, ones_m], axis=0)

    def body(i, carry):
        rowsum, colmin = carry
        At_chunk = at_ref[:, pl.ds(i * _CH, _CH)]
        d2 = jax.lax.dot_general(
            At_chunk, B, (((0,), (0,)), ((), ())),
            preferred_element_type=jnp.float32)
        rowsum = rowsum + jnp.sum(jnp.min(d2, axis=1, keepdims=True))
        colmin = jnp.minimum(colmin, jnp.min(d2, axis=0, keepdims=True))
        return rowsum, colmin

    init = (jnp.float32(0.0), jnp.full((1, _N), jnp.inf, dtype=jnp.float32))
    rowsum, colmin = jax.lax.fori_loop(0, _M // _CH, body, init)
    out_ref[...] = jnp.reshape(rowsum / _M + jnp.sum(colmin) / _N, (1, 1))


@jax.jit
def kernel(input, mask_samples):
    inpT = input.T
    yT = mask_samples[0].T
    out = pl.pallas_call(
        _chamfer_body,
        out_shape=jax.ShapeDtypeStruct((1, 1), jnp.float32),
        scratch_shapes=[pltpu.VMEM((10, _M), jnp.bfloat16)],
    )(inpT, yT)
    return out[0, 0]
